# baseline (speedup 1.0000x reference)
import functools

import jax
import jax.numpy as jnp
from jax import lax
from jax.experimental import pallas as pl
from jax.experimental.pallas import tpu as pltpu
from jax.experimental.pallas import tpu_sc as plsc

N = 4096
CTX = 4096
ACT = CTX // 2
P_CHUNKS = 64
D = 128

EMB_BLK = 256
ARG_BLK = 256

MAXSIZE = 9223372036854775807.0
INT_MAX = 2147483647


def _embed_body(x_ref, w_ref, b_ref, eap_ref, eactv_ref):
    x = x_ref[...]
    w = w_ref[...]
    b = b_ref[...]
    pooled_ap = jnp.mean(x.reshape(EMB_BLK, P_CHUNKS, CTX // P_CHUNKS), axis=-1)
    xa = x[:, :ACT]
    pooled_actv = jnp.mean(xa.reshape(EMB_BLK, P_CHUNKS, ACT // P_CHUNKS), axis=-1)
    eap_ref[...] = jnp.tanh(
        jnp.dot(pooled_ap, w, preferred_element_type=jnp.float32) + b)
    eactv_ref[...] = jnp.tanh(
        jnp.dot(pooled_actv, w, preferred_element_type=jnp.float32) + b)


def _argmin_body(erow_ref, efull_ref, hrow_ref, hcol_ref, idx_ref):
    er = erow_ref[...]
    ef = efull_ref[...]
    hr = hrow_ref[...]
    hc = hcol_ref[...]
    g = lax.dot_general(er, ef, (((1,), (1,)), ((), ())),
                        preferred_element_type=jnp.float32)
    sqr = jnp.sum(er * er, axis=1)
    sqf = jnp.sum(ef * ef, axis=1)
    d2 = (sqr[:, None] + sqf[None, :]) - 2.0 * g
    same = hr == hc
    d2 = jnp.where(same, jnp.float32(MAXSIZE), d2)
    rmin = jnp.min(d2, axis=1)
    iota = lax.broadcasted_iota(jnp.int32, (ARG_BLK, N), 1)
    cand = jnp.where(d2 == rmin[:, None], iota, jnp.int32(INT_MAX))
    idx_ref[...] = jnp.min(cand, axis=1).reshape(1, 1, ARG_BLK)


def _sc_gather(table_hbm, idx_hbm, out_hbm, idx_v, rows_v, sem):
    wid = lax.axis_index("s") * 2 + lax.axis_index("c")
    b_per_w = N // 32
    base = wid * b_per_w
    pltpu.sync_copy(idx_hbm.at[pl.ds(base, b_per_w)], idx_v)
    pltpu.async_copy(table_hbm.at[idx_v], rows_v, sem).wait()
    pltpu.sync_copy(rows_v, out_hbm.at[pl.ds(base, b_per_w)])


def kernel(context, host, W, b):
    b2 = b.reshape(1, D)
    host_i32 = host.astype(jnp.int32)

    e_ap, e_actv = pl.pallas_call(
        _embed_body,
        grid=(N // EMB_BLK,),
        in_specs=[
            pl.BlockSpec((EMB_BLK, CTX), lambda i: (i, 0)),
            pl.BlockSpec((P_CHUNKS, D), lambda i: (0, 0)),
            pl.BlockSpec((1, D), lambda i: (0, 0)),
        ],
        out_specs=[
            pl.BlockSpec((EMB_BLK, D), lambda i: (i, 0)),
            pl.BlockSpec((EMB_BLK, D), lambda i: (i, 0)),
        ],
        out_shape=[
            jax.ShapeDtypeStruct((N, D), jnp.float32),
            jax.ShapeDtypeStruct((N, D), jnp.float32),
        ],
    )(context, W, b2)

    idx3 = pl.pallas_call(
        _argmin_body,
        grid=(N // ARG_BLK,),
        in_specs=[
            pl.BlockSpec((ARG_BLK, D), lambda i: (i, 0)),
            pl.BlockSpec((N, D), lambda i: (0, 0)),
            pl.BlockSpec((ARG_BLK, 1), lambda i: (i, 0)),
            pl.BlockSpec((1, N), lambda i: (0, 0)),
        ],
        out_specs=pl.BlockSpec((1, 1, ARG_BLK), lambda i: (i, 0, 0)),
        out_shape=jax.ShapeDtypeStruct((N // ARG_BLK, 1, ARG_BLK), jnp.int32),
    )(e_actv, e_actv, host_i32.reshape(N, 1), host_i32.reshape(1, N))
    idx = idx3.reshape(N)

    mesh = plsc.VectorSubcoreMesh(core_axis_name="c", subcore_axis_name="s",
                                  num_cores=2, num_subcores=16)
    b_per_w = N // 32
    e_an = pl.kernel(
        _sc_gather,
        out_type=jax.ShapeDtypeStruct((N, D), jnp.float32),
        mesh=mesh,
        scratch_types=[
            pltpu.VMEM((b_per_w,), jnp.int32),
            pltpu.VMEM((b_per_w, D), jnp.float32),
            pltpu.SemaphoreType.DMA,
        ],
    )(e_actv, idx)

    return (e_actv, e_ap, e_an)


# speedup vs baseline: 1.9628x; 1.9628x over previous
import functools

import jax
import jax.numpy as jnp
import numpy as np
from jax import lax
from jax.experimental import pallas as pl
from jax.experimental.pallas import tpu as pltpu
from jax.experimental.pallas import tpu_sc as plsc

N = 4096
CTX = 4096
ACT = CTX // 2
P_CHUNKS = 64
D = 128

EMB_BLK = 256
ARG_BLK = 256

MAXSIZE = 9223372036854775807.0
INT_MAX = 2147483647


def _pooling_matrix():
    l = np.arange(CTX)
    pa = (l[:, None] // (CTX // P_CHUNKS) == np.arange(P_CHUNKS)[None, :])
    pa = pa.astype(np.float32) / (CTX // P_CHUNKS)
    pb = (l[:, None] // (ACT // P_CHUNKS) == np.arange(P_CHUNKS)[None, :])
    pb = (pb & (l[:, None] < ACT)).astype(np.float32) / (ACT // P_CHUNKS)
    return np.concatenate([pa, pb], axis=1)


_PPOOL = _pooling_matrix()


def _embed_body(x_ref, ppool_ref, w_ref, b_ref, eap_ref, eactv_ref):
    x = x_ref[...]
    w = w_ref[...]
    b = b_ref[...]
    pooled = jnp.dot(x, ppool_ref[...],
                     preferred_element_type=jnp.float32)
    pooled_ap = pooled[:, :P_CHUNKS]
    pooled_actv = pooled[:, P_CHUNKS:]
    eap_ref[...] = jnp.tanh(
        jnp.dot(pooled_ap, w, preferred_element_type=jnp.float32) + b)
    eactv_ref[...] = jnp.tanh(
        jnp.dot(pooled_actv, w, preferred_element_type=jnp.float32) + b)


def _argmin_body(erow_ref, efull_ref, hrow_ref, hcol_ref, idx_ref):
    er = erow_ref[...]
    ef = efull_ref[...]
    hr = hrow_ref[...]
    hc = hcol_ref[...]
    g = lax.dot_general(er, ef, (((1,), (1,)), ((), ())),
                        preferred_element_type=jnp.float32)
    sqr = jnp.sum(er * er, axis=1)
    sqf = jnp.sum(ef * ef, axis=1)
    d2 = (sqr[:, None] + sqf[None, :]) - 2.0 * g
    same = hr == hc
    d2 = jnp.where(same, jnp.float32(MAXSIZE), d2)
    rmin = jnp.min(d2, axis=1)
    iota = lax.broadcasted_iota(jnp.int32, (ARG_BLK, N), 1)
    cand = jnp.where(d2 == rmin[:, None], iota, jnp.int32(INT_MAX))
    idx_ref[...] = jnp.min(cand, axis=1).reshape(1, 1, ARG_BLK)


def _sc_gather(table_hbm, idx_hbm, out_hbm, idx_v, rows_v, sem):
    wid = lax.axis_index("s") * 2 + lax.axis_index("c")
    b_per_w = N // 32
    base = wid * b_per_w
    pltpu.sync_copy(idx_hbm.at[pl.ds(base, b_per_w)], idx_v)
    pltpu.async_copy(table_hbm.at[idx_v], rows_v, sem).wait()
    pltpu.sync_copy(rows_v, out_hbm.at[pl.ds(base, b_per_w)])


def kernel(context, host, W, b):
    b2 = b.reshape(1, D)
    host_i32 = host.astype(jnp.int32)

    e_ap, e_actv = pl.pallas_call(
        _embed_body,
        grid=(N // EMB_BLK,),
        in_specs=[
            pl.BlockSpec((EMB_BLK, CTX), lambda i: (i, 0)),
            pl.BlockSpec((CTX, 2 * P_CHUNKS), lambda i: (0, 0)),
            pl.BlockSpec((P_CHUNKS, D), lambda i: (0, 0)),
            pl.BlockSpec((1, D), lambda i: (0, 0)),
        ],
        out_specs=[
            pl.BlockSpec((EMB_BLK, D), lambda i: (i, 0)),
            pl.BlockSpec((EMB_BLK, D), lambda i: (i, 0)),
        ],
        out_shape=[
            jax.ShapeDtypeStruct((N, D), jnp.float32),
            jax.ShapeDtypeStruct((N, D), jnp.float32),
        ],
    )(context, jnp.asarray(_PPOOL), W, b2)

    idx3 = pl.pallas_call(
        _argmin_body,
        grid=(N // ARG_BLK,),
        in_specs=[
            pl.BlockSpec((ARG_BLK, D), lambda i: (i, 0)),
            pl.BlockSpec((N, D), lambda i: (0, 0)),
            pl.BlockSpec((ARG_BLK, 1), lambda i: (i, 0)),
            pl.BlockSpec((1, N), lambda i: (0, 0)),
        ],
        out_specs=pl.BlockSpec((1, 1, ARG_BLK), lambda i: (i, 0, 0)),
        out_shape=jax.ShapeDtypeStruct((N // ARG_BLK, 1, ARG_BLK), jnp.int32),
    )(e_actv, e_actv, host_i32.reshape(N, 1), host_i32.reshape(1, N))
    idx = idx3.reshape(N)

    mesh = plsc.VectorSubcoreMesh(core_axis_name="c", subcore_axis_name="s",
                                  num_cores=2, num_subcores=16)
    b_per_w = N // 32
    e_an = pl.kernel(
        _sc_gather,
        out_type=jax.ShapeDtypeStruct((N, D), jnp.float32),
        mesh=mesh,
        scratch_types=[
            pltpu.VMEM((b_per_w,), jnp.int32),
            pltpu.VMEM((b_per_w, D), jnp.float32),
            pltpu.SemaphoreType.DMA,
        ],
    )(e_actv, idx)

    return (e_actv, e_ap, e_an)
